# Initial kernel scaffold; baseline (speedup 1.0000x reference)
import jax
import jax.numpy as jnp
from jax.experimental import pallas as pl


def kernel(x, edge_index, edge_attr, batch, W1, b1, R1, W2, b2, R2, Wl1, bl1, Wl2, bl2, Wl3, bl3):
    raise NotImplementedError("write your pallas kernel here")


# speedup vs baseline: 2.6185x; 2.6185x over previous
import functools

import jax
import jax.numpy as jnp
from jax import lax
from jax.experimental import pallas as pl
from jax.experimental.pallas import tpu as pltpu
from jax.experimental.pallas import tpu_sc as plsc

N = 10000
E = 320000
F = 128
H = 512
G = 64

NC = 2
NS = 16
NW = NC * NS
EPT = E // NW
KB = 80
NBATCH = EPT // KB
APT = 640
ACC_ROWS = NS * APT
ZR = 80

RBLK = 1000
NRB = N // RBLK


def _make_segsum(ch_count):
  mesh = plsc.VectorSubcoreMesh(core_axis_name="c", subcore_axis_name="s",
                                num_cores=NC, num_subcores=NS)

  @functools.partial(
      pl.kernel,
      out_type=jax.ShapeDtypeStruct((NC, ch_count * N, F), jnp.float32),
      mesh=mesh,
      scratch_types=dict(
          src_v=pltpu.VMEM((KB,), jnp.int32),
          idx_v=pltpu.VMEM((KB,), jnp.int32),
          dst_v=pltpu.VMEM((KB,), jnp.int32),
          ew_v=pltpu.VMEM((KB,), jnp.float32),
          rows_v=pltpu.VMEM((KB, F), jnp.float32),
          stage_v=pltpu.VMEM((ZR, F), jnp.float32),
          zero_v=pltpu.VMEM((ZR, F), jnp.float32),
          acc=pltpu.VMEM_SHARED((ACC_ROWS, F), jnp.float32),
          sem_g=pltpu.SemaphoreType.DMA,
      ),
  )
  def segsum(table, src, dst, ew, out, src_v, idx_v, dst_v, ew_v, rows_v,
             stage_v, zero_v, acc, sem_g):
    c = lax.axis_index("c")
    s = lax.axis_index("s")
    wid = s * NC + c
    ebase = wid * EPT
    rowbase = s * APT

    def zfill(r, carry):
      for j in range(F // 16):
        zero_v[r, pl.ds(j * 16, 16)] = jnp.zeros((16,), jnp.float32)
      return carry
    lax.fori_loop(0, ZR, zfill, 0)

    for ch in range(ch_count):
      for z in range(APT // ZR):
        pltpu.sync_copy(zero_v, acc.at[pl.ds(rowbase + z * ZR, ZR)])
      plsc.subcore_barrier()

      def batch_body(b, carry):
        off = ebase + b * KB
        pltpu.sync_copy(src.at[pl.ds(off, KB)], src_v)
        pltpu.sync_copy(dst.at[pl.ds(off, KB)], dst_v)
        pltpu.sync_copy(ew.at[pl.ds(off, KB)], ew_v)
        if ch_count == 1:
          gidx = src_v
        else:
          for i in range(KB // 16):
            sl = pl.ds(i * 16, 16)
            idx_v[sl] = src_v[sl] * ch_count + ch
          gidx = idx_v
        pltpu.async_copy(table.at[gidx], rows_v, sem_g).wait()

        def mul_body(g, mc):
          ew16 = ew_v[pl.ds(pl.multiple_of(g * 16, 16), 16)]
          for l in range(16):
            k = g * 16 + l
            w = ew16[l]
            for j in range(F // 16):
              sl = pl.ds(j * 16, 16)
              rows_v[k, sl] = rows_v[k, sl] * w
          return mc
        lax.fori_loop(0, KB // 16, mul_body, 0)

        pltpu.sync_copy(rows_v, acc.at[dst_v], add=True)
        return carry
      lax.fori_loop(0, NBATCH, batch_body, 0)
      plsc.subcore_barrier()

      for z in range(APT // ZR):
        r0 = pl.multiple_of(rowbase + z * ZR, ZR)

        @pl.when(r0 < N)
        def _():
          pltpu.sync_copy(acc.at[pl.ds(r0, ZR)], stage_v)
          pltpu.sync_copy(stage_v, out.at[c, pl.ds(ch * N + r0, ZR)])
      if ch != ch_count - 1:
        plsc.subcore_barrier()

  return segsum


_segsum1 = _make_segsum(1)
_segsum4 = _make_segsum(H // F)


def _tc1_body(aggp_ref, x_ref, w1_ref, b1_ref, r1_ref, o_ref):
  agg = aggp_ref[0] + aggp_ref[1]
  o_ref[...] = jnp.maximum(
      jnp.dot(agg, w1_ref[...], preferred_element_type=jnp.float32)
      + b1_ref[...]
      + jnp.dot(x_ref[...], r1_ref[...], preferred_element_type=jnp.float32),
      0.0)


def _tc1(aggp, x, w1, b1, r1):
  return pl.pallas_call(
      _tc1_body,
      grid=(NRB,),
      in_specs=[
          pl.BlockSpec((NC, RBLK, F), lambda i: (0, i, 0)),
          pl.BlockSpec((RBLK, F), lambda i: (i, 0)),
          pl.BlockSpec((F, H), lambda i: (0, 0)),
          pl.BlockSpec((1, H), lambda i: (0, 0)),
          pl.BlockSpec((F, H), lambda i: (0, 0)),
      ],
      out_specs=pl.BlockSpec((RBLK, H), lambda i: (i, 0)),
      out_shape=jax.ShapeDtypeStruct((N, H), jnp.float32),
  )(aggp, x, w1, b1, r1)


def _tc2_body(aggp_ref, h1_ref, batch_ref, w2_ref, b2_ref, r2_ref,
              wl1_ref, bl1_ref, wl2_ref, bl2_ref, wl3_ref, bl3_ref,
              o_ref, acc_ref, pooled_ref, cnt_ref):
  i = pl.program_id(0)
  c = pl.program_id(1)
  nchunk = H // F

  @pl.when(jnp.logical_and(i == 0, c == 0))
  def _():
    pooled_ref[...] = jnp.zeros_like(pooled_ref)
    cnt_ref[...] = jnp.zeros_like(cnt_ref)

  @pl.when(c == 0)
  def _():
    acc_ref[...] = jnp.zeros_like(acc_ref)

  aggc = aggp_ref[0] + aggp_ref[1]
  acc_ref[...] += (
      jnp.dot(aggc, w2_ref[...], preferred_element_type=jnp.float32)
      + jnp.dot(h1_ref[...], r2_ref[...], preferred_element_type=jnp.float32))

  @pl.when(c == nchunk - 1)
  def _():
    h2 = jnp.maximum(acc_ref[...] + b2_ref[...], 0.0)
    brow = batch_ref[0, 0, :]
    gids = lax.broadcasted_iota(jnp.int32, (G, RBLK), 0)
    onehot = (gids == brow[None, :]).astype(jnp.float32)
    pooled_ref[...] += jnp.dot(onehot, h2, preferred_element_type=jnp.float32)
    cnt_ref[...] += jnp.broadcast_to(
        jnp.sum(onehot, axis=1, keepdims=True), cnt_ref.shape)

  @pl.when(jnp.logical_and(i == NRB - 1, c == nchunk - 1))
  def _():
    cnt = jnp.maximum(cnt_ref[...][:, :1], 1.0)
    pooled = pooled_ref[...] / cnt
    z = jnp.maximum(
        jnp.dot(pooled, wl1_ref[...], preferred_element_type=jnp.float32)
        + bl1_ref[...], 0.0)
    z = jnp.maximum(
        jnp.dot(z, wl2_ref[...], preferred_element_type=jnp.float32)
        + bl2_ref[...], 0.0)
    r = jnp.sum(z * wl3_ref[...], axis=1, keepdims=True) + bl3_ref[...]
    o_ref[...] = jnp.broadcast_to(r, o_ref.shape)


def _tc2(aggp, h1, batch3, w2, b2, r2, wl1, bl1, wl2, bl2, wl3r, bl3r):
  nchunk = H // F
  return pl.pallas_call(
      _tc2_body,
      grid=(NRB, nchunk),
      in_specs=[
          pl.BlockSpec((NC, RBLK, F), lambda i, c: (0, c * NRB + i, 0)),
          pl.BlockSpec((RBLK, F), lambda i, c: (i, c)),
          pl.BlockSpec((1, 1, RBLK), lambda i, c: (i, 0, 0)),
          pl.BlockSpec((F, H), lambda i, c: (c, 0)),
          pl.BlockSpec((1, H), lambda i, c: (0, 0)),
          pl.BlockSpec((F, H), lambda i, c: (c, 0)),
          pl.BlockSpec((H, G), lambda i, c: (0, 0)),
          pl.BlockSpec((1, G), lambda i, c: (0, 0)),
          pl.BlockSpec((G, 16), lambda i, c: (0, 0)),
          pl.BlockSpec((1, 16), lambda i, c: (0, 0)),
          pl.BlockSpec((1, 16), lambda i, c: (0, 0)),
          pl.BlockSpec((1, 1), lambda i, c: (0, 0)),
      ],
      out_specs=pl.BlockSpec((G, F), lambda i, c: (0, 0)),
      out_shape=jax.ShapeDtypeStruct((G, F), jnp.float32),
      scratch_shapes=[
          pltpu.VMEM((RBLK, H), jnp.float32),
          pltpu.VMEM((G, H), jnp.float32),
          pltpu.VMEM((G, F), jnp.float32),
      ],
  )(aggp, h1, batch3, w2, b2, r2, wl1, bl1, wl2, bl2, wl3r, bl3r)


def kernel(x, edge_index, edge_attr, batch, W1, b1, R1, W2, b2, R2,
           Wl1, bl1, Wl2, bl2, Wl3, bl3):
  src = edge_index[0]
  dst = edge_index[1]
  agg1p = _segsum1(x, src, dst, edge_attr)
  h1 = _tc1(agg1p, x, W1, b1.reshape(1, H), R1)
  agg2p = _segsum4(h1.reshape((H // F) * N, F), src, dst, edge_attr)
  out = _tc2(agg2p, h1, batch.reshape(NRB, 1, RBLK), W2, b2.reshape(1, H), R2,
             Wl1, bl1.reshape(1, G), Wl2, bl2.reshape(1, 16),
             Wl3.reshape(1, 16), bl3.reshape(1, 1))
  return out[:, :1]


# speedup vs baseline: 5.0500x; 1.9286x over previous
import functools

import jax
import jax.numpy as jnp
from jax import lax
from jax.experimental import pallas as pl
from jax.experimental.pallas import tpu as pltpu
from jax.experimental.pallas import tpu_sc as plsc

N = 10000
E = 320000
F = 128
H = 512
G = 64

NC = 2
NS = 16
NW = NC * NS
EPT = E // NW
KB = 80
NBATCH = EPT // KB
APT = 640
ACC_ROWS = NS * APT
ZR = 80

RBLK = 1000
NRB = N // RBLK


def _make_segsum(ch_count):
  mesh = plsc.VectorSubcoreMesh(core_axis_name="c", subcore_axis_name="s",
                                num_cores=NC, num_subcores=NS)

  @functools.partial(
      pl.kernel,
      out_type=jax.ShapeDtypeStruct((NC, ch_count * N, F), jnp.float32),
      mesh=mesh,
      scratch_types=dict(
          srcb0=pltpu.VMEM((KB,), jnp.int32),
          srcb1=pltpu.VMEM((KB,), jnp.int32),
          dstb0=pltpu.VMEM((KB,), jnp.int32),
          dstb1=pltpu.VMEM((KB,), jnp.int32),
          ewb0=pltpu.VMEM((KB,), jnp.float32),
          ewb1=pltpu.VMEM((KB,), jnp.float32),
          idx0_v=pltpu.VMEM((KB,), jnp.int32),
          idx1_v=pltpu.VMEM((KB,), jnp.int32),
          rows0_v=pltpu.VMEM((KB, F), jnp.float32),
          rows1_v=pltpu.VMEM((KB, F), jnp.float32),
          acc=pltpu.VMEM_SHARED((ACC_ROWS, F), jnp.float32),
          sem0=pltpu.SemaphoreType.DMA,
          sem1=pltpu.SemaphoreType.DMA,
          sem_e0=pltpu.SemaphoreType.DMA,
          sem_e1=pltpu.SemaphoreType.DMA,
      ),
  )
  def segsum(table, src, dst, ew, out, srcb0, srcb1, dstb0, dstb1, ewb0,
             ewb1, idx0_v, idx1_v, rows0_v, rows1_v, acc, sem0, sem1,
             sem_e0, sem_e1):
    c = lax.axis_index("c")
    s = lax.axis_index("s")
    wid = s * NC + c
    ebase = pl.multiple_of(wid * EPT, EPT)
    rowbase = s * APT
    rows = (rows0_v, rows1_v)
    idxs = (idx0_v, idx1_v)
    sems = (sem0, sem1)
    srcb = (srcb0, srcb1)
    dstb = (dstb0, dstb1)
    ewb = (ewb0, ewb1)
    sems_e = (sem_e0, sem_e1)

    def issue_edges(b, bufi):
      off = pl.multiple_of(ebase + b * KB, KB)
      pltpu.async_copy(src.at[pl.ds(off, KB)], srcb[bufi], sems_e[bufi])
      pltpu.async_copy(dst.at[pl.ds(off, KB)], dstb[bufi], sems_e[bufi])
      pltpu.async_copy(ew.at[pl.ds(off, KB)], ewb[bufi], sems_e[bufi])

    def wait_edges(bufi):
      off = pl.multiple_of(ebase, KB)
      pltpu.make_async_copy(src.at[pl.ds(off, KB)], srcb[bufi],
                            sems_e[bufi]).wait()
      pltpu.make_async_copy(dst.at[pl.ds(off, KB)], dstb[bufi],
                            sems_e[bufi]).wait()
      pltpu.make_async_copy(ew.at[pl.ds(off, KB)], ewb[bufi],
                            sems_e[bufi]).wait()

    def issue_gather(ch, bufi):
      gidx = idxs[bufi]
      for i in range(KB // 16):
        sl = pl.ds(i * 16, 16)
        if ch_count == 1:
          gidx[sl] = srcb[bufi][sl]
        else:
          gidx[sl] = srcb[bufi][sl] * ch_count + ch
      return pltpu.async_copy(table.at[gidx], rows[bufi], sems[bufi])

    def multiply(bufi):
      rv = rows[bufi]
      ev = ewb[bufi]

      def mul_body(g, mc):
        ew16 = ev[pl.ds(pl.multiple_of(g * 16, 16), 16)]
        for l in range(16):
          k = g * 16 + l
          w = ew16[l]
          for j in range(F // 16):
            sl = pl.ds(j * 16, 16)
            rv[k, sl] = rv[k, sl] * w
        return mc
      lax.fori_loop(0, KB // 16, mul_body, 0)

    def issue_scatter(bufi):
      return pltpu.async_copy(rows[bufi], acc.at[dstb[bufi]], sems[bufi],
                              add=True)

    def wait_gather(bufi):
      pltpu.make_async_copy(table.at[idxs[bufi]], rows[bufi],
                            sems[bufi]).wait()

    def wait_scatter(bufi):
      pltpu.make_async_copy(rows[bufi], acc.at[dstb[bufi]],
                            sems[bufi]).wait()

    def chunk_body(ch, carry):
      def zfill(r, carry2):
        for j in range(F // 16):
          rows1_v[r, pl.ds(j * 16, 16)] = jnp.zeros((16,), jnp.float32)
        return carry2
      lax.fori_loop(0, KB, zfill, 0)

      for z in range(APT // ZR):
        pltpu.sync_copy(rows1_v, acc.at[pl.ds(rowbase + z * ZR, ZR)])
      plsc.subcore_barrier()

      issue_edges(jnp.int32(0), 0)
      wait_edges(0)
      issue_gather(ch, 0)
      pltpu.async_copy(rows1_v, acc.at[dstb0], sem1, add=True)

      def half_step(b, bufi, last):
        other = 1 - bufi
        wait_scatter(other)
        if not last:
          issue_edges(b + 1, other)
        wait_gather(bufi)
        multiply(bufi)
        issue_scatter(bufi)
        if not last:
          wait_edges(other)
          issue_gather(ch, other)

      def pair_body(p, pc):
        half_step(2 * p, 0, False)
        half_step(2 * p + 1, 1, False)
        return pc
      lax.fori_loop(0, (NBATCH - 1) // 2, pair_body, 0)
      half_step(jnp.int32(NBATCH - 1), 0, True)
      wait_scatter(0)
      plsc.subcore_barrier()

      for z in range(APT // ZR):
        r0 = pl.multiple_of(rowbase + z * ZR, ZR)

        @pl.when(r0 < N)
        def _():
          pltpu.sync_copy(acc.at[pl.ds(r0, ZR)], out.at[c, pl.ds(ch * N + r0, ZR)])
      plsc.subcore_barrier()
      return carry

    lax.fori_loop(0, ch_count, chunk_body, 0)

  return segsum


_segsum1 = _make_segsum(1)
_segsum4 = _make_segsum(H // F)


def _tc1_body(aggp_ref, x_ref, w1_ref, b1_ref, r1_ref, o_ref):
  agg = aggp_ref[0] + aggp_ref[1]
  o_ref[...] = jnp.maximum(
      jnp.dot(agg, w1_ref[...], preferred_element_type=jnp.float32)
      + b1_ref[...]
      + jnp.dot(x_ref[...], r1_ref[...], preferred_element_type=jnp.float32),
      0.0)


def _tc1(aggp, x, w1, b1, r1):
  return pl.pallas_call(
      _tc1_body,
      grid=(NRB,),
      in_specs=[
          pl.BlockSpec((NC, RBLK, F), lambda i: (0, i, 0)),
          pl.BlockSpec((RBLK, F), lambda i: (i, 0)),
          pl.BlockSpec((F, H), lambda i: (0, 0)),
          pl.BlockSpec((1, H), lambda i: (0, 0)),
          pl.BlockSpec((F, H), lambda i: (0, 0)),
      ],
      out_specs=pl.BlockSpec((RBLK, H), lambda i: (i, 0)),
      out_shape=jax.ShapeDtypeStruct((N, H), jnp.float32),
  )(aggp, x, w1, b1, r1)


def _tc2_body(aggp_ref, h1_ref, batch_ref, w2_ref, b2_ref, r2_ref,
              wl1_ref, bl1_ref, wl2_ref, bl2_ref, wl3_ref, bl3_ref,
              o_ref, acc_ref, pooled_ref, cnt_ref):
  i = pl.program_id(0)
  c = pl.program_id(1)
  nchunk = H // F

  @pl.when(jnp.logical_and(i == 0, c == 0))
  def _():
    pooled_ref[...] = jnp.zeros_like(pooled_ref)
    cnt_ref[...] = jnp.zeros_like(cnt_ref)

  @pl.when(c == 0)
  def _():
    acc_ref[...] = jnp.zeros_like(acc_ref)

  aggc = aggp_ref[0] + aggp_ref[1]
  acc_ref[...] += (
      jnp.dot(aggc, w2_ref[...], preferred_element_type=jnp.float32)
      + jnp.dot(h1_ref[...], r2_ref[...], preferred_element_type=jnp.float32))

  @pl.when(c == nchunk - 1)
  def _():
    h2 = jnp.maximum(acc_ref[...] + b2_ref[...], 0.0)
    brow = batch_ref[0, 0, :]
    gids = lax.broadcasted_iota(jnp.int32, (G, RBLK), 0)
    onehot = (gids == brow[None, :]).astype(jnp.float32)
    pooled_ref[...] += jnp.dot(onehot, h2, preferred_element_type=jnp.float32)
    cnt_ref[...] += jnp.broadcast_to(
        jnp.sum(onehot, axis=1, keepdims=True), cnt_ref.shape)

  @pl.when(jnp.logical_and(i == NRB - 1, c == nchunk - 1))
  def _():
    cnt = jnp.maximum(cnt_ref[...][:, :1], 1.0)
    pooled = pooled_ref[...] / cnt
    z = jnp.maximum(
        jnp.dot(pooled, wl1_ref[...], preferred_element_type=jnp.float32)
        + bl1_ref[...], 0.0)
    z = jnp.maximum(
        jnp.dot(z, wl2_ref[...], preferred_element_type=jnp.float32)
        + bl2_ref[...], 0.0)
    r = jnp.sum(z * wl3_ref[...], axis=1, keepdims=True) + bl3_ref[...]
    o_ref[...] = jnp.broadcast_to(r, o_ref.shape)


def _tc2(aggp, h1, batch3, w2, b2, r2, wl1, bl1, wl2, bl2, wl3r, bl3r):
  nchunk = H // F
  return pl.pallas_call(
      _tc2_body,
      grid=(NRB, nchunk),
      in_specs=[
          pl.BlockSpec((NC, RBLK, F), lambda i, c: (0, c * NRB + i, 0)),
          pl.BlockSpec((RBLK, F), lambda i, c: (i, c)),
          pl.BlockSpec((1, 1, RBLK), lambda i, c: (i, 0, 0)),
          pl.BlockSpec((F, H), lambda i, c: (c, 0)),
          pl.BlockSpec((1, H), lambda i, c: (0, 0)),
          pl.BlockSpec((F, H), lambda i, c: (c, 0)),
          pl.BlockSpec((H, G), lambda i, c: (0, 0)),
          pl.BlockSpec((1, G), lambda i, c: (0, 0)),
          pl.BlockSpec((G, 16), lambda i, c: (0, 0)),
          pl.BlockSpec((1, 16), lambda i, c: (0, 0)),
          pl.BlockSpec((1, 16), lambda i, c: (0, 0)),
          pl.BlockSpec((1, 1), lambda i, c: (0, 0)),
      ],
      out_specs=pl.BlockSpec((G, F), lambda i, c: (0, 0)),
      out_shape=jax.ShapeDtypeStruct((G, F), jnp.float32),
      scratch_shapes=[
          pltpu.VMEM((RBLK, H), jnp.float32),
          pltpu.VMEM((G, H), jnp.float32),
          pltpu.VMEM((G, F), jnp.float32),
      ],
  )(aggp, h1, batch3, w2, b2, r2, wl1, bl1, wl2, bl2, wl3r, bl3r)


def kernel(x, edge_index, edge_attr, batch, W1, b1, R1, W2, b2, R2,
           Wl1, bl1, Wl2, bl2, Wl3, bl3):
  src = edge_index[0]
  dst = edge_index[1]
  agg1p = _segsum1(x, src, dst, edge_attr)
  h1 = _tc1(agg1p, x, W1, b1.reshape(1, H), R1)
  agg2p = _segsum4(h1.reshape((H // F) * N, F), src, dst, edge_attr)
  out = _tc2(agg2p, h1, batch.reshape(NRB, 1, RBLK), W2, b2.reshape(1, H), R2,
             Wl1, bl1.reshape(1, G), Wl2, bl2.reshape(1, 16),
             Wl3.reshape(1, 16), bl3.reshape(1, 1))
  return out[:, :1]
